# baseline (speedup 1.0000x reference)
import functools

import jax
import jax.numpy as jnp
from jax import lax
from jax.experimental import pallas as pl
from jax.experimental.pallas import tpu as pltpu
from jax.experimental.pallas import tpu_sc as plsc

VOCAB = 100000
D = 64
N = 4096 * 200

_INFO = plsc.get_sparse_core_info()
NC = _INFO.num_cores
NS = _INFO.num_subcores
NW = NC * NS

PER_W = N // NW
ROWS = 400
HROWS = ROWS // 2
NBUF = 4
CHUNKS = PER_W // ROWS
GROUPS = CHUNKS // NBUF
OUT_N = N * D // 128


def _body(xfe_hbm, xfo_hbm, xte_hbm, xto_hbm, wm_hbm, tt_hbm, out_hbm,
          idx_v, rows_v, sems):
    wid = lax.axis_index("s") * NC + lax.axis_index("c")
    hbase = pl.multiple_of(wid * (PER_W // 2), PER_W // 2)

    def stage_idx(g, b):
        off = hbase + (g * NBUF + b) * HROWS
        srcs = (xfe_hbm, xfo_hbm, xte_hbm, xto_hbm)
        return [pltpu.make_async_copy(src.at[pl.ds(off, HROWS)],
                                      idx_v.at[i, b], sems.at[b])
                for i, src in enumerate(srcs)]

    def gathers(b, table, i0):
        tbl = (wm_hbm, tt_hbm)[table]
        return (pltpu.make_async_copy(tbl.at[idx_v.at[i0, b]],
                                      rows_v.at[b, 0],
                                      sems.at[b]),
                pltpu.make_async_copy(tbl.at[idx_v.at[i0 + 1, b]],
                                      rows_v.at[b, 1],
                                      sems.at[b]))

    def scatter_out(g, b):
        off = hbase + (g * NBUF + b) * HROWS
        return (pltpu.make_async_copy(rows_v.at[b, 0],
                                      out_hbm.at[pl.ds(off, HROWS),
                                                 pl.ds(0, D)],
                                      sems.at[b]),
                pltpu.make_async_copy(rows_v.at[b, 1],
                                      out_hbm.at[pl.ds(off, HROWS),
                                                 pl.ds(D, D)],
                                      sems.at[b]))

    def group_body(g, carry):
        for b in range(NBUF):
            @pl.when(g > 0)
            def _wait_prev():
                for cp in scatter_out(g - 1, b):
                    cp.wait()
            for cp in stage_idx(g, b):
                cp.start()
        for b in range(NBUF):
            for cp in stage_idx(g, b):
                cp.wait()
            ge, go = gathers(b, 0, 0)
            ge.start()
            go.start()
        for b in range(NBUF):
            ge, go = gathers(b, 0, 0)
            ge.wait()
            go.wait()
            ae, ao = gathers(b, 1, 2)
            ae.start(add=True)
            ao.start(add=True)
        for b in range(NBUF):
            ae, ao = gathers(b, 1, 2)
            ae.wait()
            ao.wait()
            for cp in scatter_out(g, b):
                cp.start()
        return carry

    lax.fori_loop(0, GROUPS, group_body, 0)
    for b in range(NBUF):
        for cp in scatter_out(GROUPS - 1, b):
            cp.wait()


@jax.jit
def _dual_embed(xfe, xfo, xte, xto, wm, tt):
    mesh = plsc.VectorSubcoreMesh(core_axis_name="c", subcore_axis_name="s")
    f = functools.partial(
        pl.kernel,
        out_type=jax.ShapeDtypeStruct((OUT_N, 128), jnp.float32),
        mesh=mesh,
        scratch_types=[
            pltpu.VMEM((4, NBUF, HROWS), jnp.int32),
            pltpu.VMEM((NBUF, 2, HROWS, D), jnp.float32),
            pltpu.SemaphoreType.DMA((NBUF,)),
        ],
        compiler_params=pltpu.CompilerParams(use_tc_tiling_on_sc=False),
    )(_body)
    return f(xfe, xfo, xte, xto, wm, tt)


def kernel(x_fix, x_train, word_mat, trained_table):
    b, s = x_fix.shape
    xf = x_fix.reshape(-1, 2).astype(jnp.int32)
    xt = x_train.reshape(-1, 2).astype(jnp.int32)
    out = _dual_embed(xf[:, 0], xf[:, 1], xt[:, 0], xt[:, 1],
                      word_mat, trained_table)
    return out.reshape(b, s, D)


# speedup vs baseline: 2.5407x; 2.5407x over previous
import functools

import jax
import jax.numpy as jnp
from jax import lax
from jax.experimental import pallas as pl
from jax.experimental.pallas import tpu as pltpu
from jax.experimental.pallas import tpu_sc as plsc

VOCAB = 100000
D = 64
B = 4096
S = 200

_INFO = plsc.get_sparse_core_info()
NC = _INFO.num_cores
NS = _INFO.num_subcores
NW = NC * NS

B_W = B // NW
NBUF = 4
GROUPS = B_W // NBUF


def _body(xf_hbm, xt_hbm, wm_hbm, tt_hbm, out_hbm, idxf_v, idxt_v, rows_v, sems):
    wid = lax.axis_index("s") * NC + lax.axis_index("c")
    base = pl.multiple_of(wid * B_W * S, B_W * S)

    pltpu.sync_copy(xf_hbm.at[pl.ds(base, B_W * S)], idxf_v)
    pltpu.sync_copy(xt_hbm.at[pl.ds(base, B_W * S)], idxt_v)

    def gather_a(g, b):
        off = (g * NBUF + b) * S
        return pltpu.make_async_copy(wm_hbm.at[idxf_v.at[pl.ds(off, S)]],
                                     rows_v.at[b], sems.at[b])

    def gather_b(g, b):
        off = (g * NBUF + b) * S
        return pltpu.make_async_copy(tt_hbm.at[idxt_v.at[pl.ds(off, S)]],
                                     rows_v.at[b], sems.at[b])

    def scatter_out(g, b):
        bi = wid * B_W + g * NBUF + b
        return pltpu.make_async_copy(rows_v.at[b],
                                     out_hbm.at[bi, :, pl.ds(0, D)],
                                     sems.at[b])

    def group_body(g, carry):
        for b in range(NBUF):
            @pl.when(g > 0)
            def _wait_prev():
                scatter_out(g - 1, b).wait()
            gather_a(g, b).start()
        for b in range(NBUF):
            gather_a(g, b).wait()
            gather_b(g, b).start(add=True)
        for b in range(NBUF):
            gather_b(g, b).wait()
            scatter_out(g, b).start()
        return carry

    lax.fori_loop(0, GROUPS, group_body, 0)
    for b in range(NBUF):
        scatter_out(GROUPS - 1, b).wait()


@jax.jit
def _dual_embed(xf, xt, wm, tt):
    mesh = plsc.VectorSubcoreMesh(core_axis_name="c", subcore_axis_name="s")
    f = functools.partial(
        pl.kernel,
        out_type=jax.ShapeDtypeStruct((B, S, 128), jnp.float32),
        mesh=mesh,
        scratch_types=[
            pltpu.VMEM((B_W * S,), jnp.int32),
            pltpu.VMEM((B_W * S,), jnp.int32),
            pltpu.VMEM((NBUF, S, D), jnp.float32),
            pltpu.SemaphoreType.DMA((NBUF,)),
        ],
        compiler_params=pltpu.CompilerParams(use_tc_tiling_on_sc=False),
    )(_body)
    return f(xf, xt, wm, tt)


def kernel(x_fix, x_train, word_mat, trained_table):
    b, s = x_fix.shape
    xf = x_fix.reshape(-1).astype(jnp.int32)
    xt = x_train.reshape(-1).astype(jnp.int32)
    out = _dual_embed(xf, xt, word_mat, trained_table)
    return out[:, :, :D]
